# baseline (speedup 1.0000x reference)
import functools

import jax
import jax.numpy as jnp
from jax import lax
from jax.experimental import pallas as pl
from jax.experimental.pallas import tpu as pltpu
from jax.experimental.pallas import tpu_sc as plsc

B = 16384
D = 32
RPG = 128 // D
NC, NS, L = 2, 16, 16
NW = NC * NS
BPW = B // NW
CHUNK = 128
NCHUNK = BPW // CHUNK
GPC = CHUNK // L


def _sc_dots(u, i, j, W2, H2):
    mesh = plsc.VectorSubcoreMesh(core_axis_name="c", subcore_axis_name="s")

    @functools.partial(
        pl.kernel,
        out_type=jax.ShapeDtypeStruct((B,), jnp.float32),
        mesh=mesh,
        scratch_types=[
            pltpu.VMEM((BPW,), jnp.int32),
            pltpu.VMEM((BPW,), jnp.int32),
            pltpu.VMEM((BPW,), jnp.int32),
            pltpu.VMEM((BPW,), jnp.int32),
            pltpu.VMEM((BPW,), jnp.int32),
            pltpu.VMEM((BPW,), jnp.int32),
            pltpu.VMEM((BPW,), jnp.int32),
            pltpu.VMEM((BPW,), jnp.int32),
            pltpu.VMEM((BPW,), jnp.int32),
            pltpu.VMEM((CHUNK, 128), jnp.float32),
            pltpu.VMEM((CHUNK, 128), jnp.float32),
            pltpu.VMEM((CHUNK, 128), jnp.float32),
            pltpu.VMEM((BPW,), jnp.float32),
            pltpu.SemaphoreType.DMA,
        ],
        compiler_params=pltpu.CompilerParams(needs_layout_passes=False),
    )
    def k(u_hbm, i_hbm, j_hbm, W_hbm, H_hbm, out_hbm,
          idx_u, idx_i, idx_j, tid_u, tid_i, tid_j, col_u, col_i, col_j,
          u_buf, i_buf, j_buf, x_out, sem):
        wid = lax.axis_index("s") * NC + lax.axis_index("c")
        base = pl.multiple_of(wid * BPW, BPW)

        pltpu.sync_copy(u_hbm.at[pl.ds(base, BPW)], idx_u)
        pltpu.sync_copy(i_hbm.at[pl.ds(base, BPW)], idx_i)
        pltpu.sync_copy(j_hbm.at[pl.ds(base, BPW)], idx_j)

        def split(k_, carry):
            sl = pl.ds(k_ * L, L)
            for idx_r, tid_r, col_r in ((idx_u, tid_u, col_u),
                                        (idx_i, tid_i, col_i),
                                        (idx_j, tid_j, col_j)):
                v = idx_r[sl]
                tid_r[sl] = lax.shift_right_logical(v, 2)
                col_r[sl] = lax.shift_left(jnp.bitwise_and(v, 3), 5)
            return carry

        lax.fori_loop(0, BPW // L, split, 0)

        lanes = lax.iota(jnp.int32, L)

        for c in range(NCHUNK):
            cs = pl.ds(c * CHUNK, CHUNK)
            descs = [
                pltpu.async_copy(W_hbm.at[tid_u.at[cs]], u_buf, sem),
                pltpu.async_copy(H_hbm.at[tid_i.at[cs]], i_buf, sem),
                pltpu.async_copy(H_hbm.at[tid_j.at[cs]], j_buf, sem),
            ]
            for dsc in descs:
                dsc.wait()

            def body(g, carry):
                rows = g * L + lanes
                gsl = pl.ds(c * CHUNK + g * L, L)
                cu0 = col_u[gsl]
                ci0 = col_i[gsl]
                cj0 = col_j[gsl]
                acc_ui = jnp.zeros((L,), jnp.float32)
                acc_uj = jnp.zeros((L,), jnp.float32)
                for d in range(D):
                    uv = plsc.load_gather(u_buf, [rows, cu0 + d])
                    iv = plsc.load_gather(i_buf, [rows, ci0 + d])
                    jv = plsc.load_gather(j_buf, [rows, cj0 + d])
                    acc_ui = acc_ui + uv * iv
                    acc_uj = acc_uj + uv * jv
                x_out[gsl] = acc_ui - acc_uj
                return carry

            lax.fori_loop(0, GPC, body, 0)

        pltpu.sync_copy(x_out, out_hbm.at[pl.ds(base, BPW)])

    return k(u, i, j, W2, H2)


def _neg_logsig_sum(x):
    def body(x_ref, o_ref):
        v = x_ref[...]
        sp = jnp.maximum(-v, 0.0) + jnp.log(1.0 + jnp.exp(-jnp.abs(v)))
        o_ref[0, 0] = jnp.sum(sp)

    out = pl.pallas_call(
        body,
        out_shape=jax.ShapeDtypeStruct((1, 1), jnp.float32),
        out_specs=pl.BlockSpec(memory_space=pltpu.SMEM),
    )(x.reshape(128, 128))
    return out[0, 0]


def kernel(u, i, j, W, H):
    W2 = W.reshape(W.shape[0] // RPG, RPG * D)
    H2 = H.reshape(H.shape[0] // RPG, RPG * D)
    x = _sc_dots(u.astype(jnp.int32), i.astype(jnp.int32), j.astype(jnp.int32),
                 W2, H2)
    return _neg_logsig_sum(x)
